# baseline (speedup 1.0000x reference)
import functools

import jax
import jax.numpy as jnp
from jax import lax
from jax.experimental import pallas as pl
from jax.experimental.pallas import tpu as pltpu
from jax.experimental.pallas import tpu_sc as plsc

N = 10000
E = 320000
D = 128
R = 8
B = 4

NPAD = 10240
NC = 2
NS = 16
NW = NC * NS
C = 128
K = 2 * (-(-E // (NW * C * 2)))
EPW = K * C
EPAD = NW * EPW

BN = 2048
NB = NPAD // BN


def _table_body(comp_ref, x_ref, v_ref, out_ref):
    r = pl.program_id(1)
    w = (comp_ref[r, 0] * v_ref[0]
         + comp_ref[r, 1] * v_ref[1]
         + comp_ref[r, 2] * v_ref[2]
         + comp_ref[r, 3] * v_ref[3])
    out_ref[...] = jnp.dot(x_ref[...], w, preferred_element_type=jnp.float32)


def _make_table(x, v, comp):
    return pl.pallas_call(
        _table_body,
        grid=(NB, R),
        in_specs=[
            pl.BlockSpec(memory_space=pltpu.SMEM),
            pl.BlockSpec((BN, D), lambda i, r: (i, 0)),
            pl.BlockSpec((B, D, D), lambda i, r: (0, 0, 0)),
        ],
        out_specs=pl.BlockSpec((BN, D), lambda i, r: (r * NB + i, 0)),
        out_shape=jax.ShapeDtypeStruct((R * NPAD, D), jnp.float32),
    )(comp, x, v)


@functools.cache
def _sc_edges_fn():
    mesh = plsc.VectorSubcoreMesh(
        core_axis_name="c", subcore_axis_name="s",
        num_cores=NC, num_subcores=NS)

    @functools.partial(
        pl.kernel,
        out_type=jax.ShapeDtypeStruct((NC * NPAD, D), jnp.float32),
        mesh=mesh,
        scratch_types=[
            pltpu.VMEM((K, C), jnp.int32),
            pltpu.VMEM((K, C), jnp.int32),
            pltpu.VMEM((C, D), jnp.float32),
            pltpu.VMEM_SHARED((NPAD, D), jnp.float32),
            pltpu.SemaphoreType.DMA,
        ],
    )
    def _sc_edges(table_hbm, gidx_hbm, didx_hbm, zeros_hbm, out_hbm,
                  gidx_v, didx_v, rows0, acc, gs0):
        cid = lax.axis_index("c")
        sid = lax.axis_index("s")
        wid = cid * NS + sid
        stripe = NPAD // NS

        pltpu.sync_copy(zeros_hbm.at[pl.ds(sid * stripe, stripe)],
                        acc.at[pl.ds(sid * stripe, stripe)])
        pltpu.sync_copy(gidx_hbm.at[wid], gidx_v)
        pltpu.sync_copy(didx_hbm.at[wid], didx_v)
        plsc.subcore_barrier()

        def chunk(g, carry):
            pltpu.async_copy(table_hbm.at[gidx_v.at[g]], rows0, gs0).wait()
            pltpu.sync_copy(rows0, acc.at[didx_v.at[g]], add=True)
            return carry

        lax.fori_loop(0, K, chunk, 0)
        plsc.subcore_barrier()

        pltpu.sync_copy(acc.at[pl.ds(sid * stripe, stripe)],
                        out_hbm.at[pl.ds(cid * NPAD + sid * stripe, stripe)])

    return _sc_edges


def _combine_body(p0_ref, p1_ref, x_ref, w_ref, b_ref, out_ref, *, relu):
    h = (p0_ref[...] + p1_ref[...] + b_ref[...]
         + jnp.dot(x_ref[...], w_ref[...], preferred_element_type=jnp.float32))
    out_ref[...] = jnp.maximum(h, 0.0) if relu else h


def _combine(p, x, wself, b, relu):
    return pl.pallas_call(
        functools.partial(_combine_body, relu=relu),
        grid=(NB,),
        in_specs=[
            pl.BlockSpec((BN, D), lambda i: (i, 0)),
            pl.BlockSpec((BN, D), lambda i: (NB + i, 0)),
            pl.BlockSpec((BN, D), lambda i: (i, 0)),
            pl.BlockSpec((D, D), lambda i: (0, 0)),
            pl.BlockSpec((1, D), lambda i: (0, 0)),
        ],
        out_specs=pl.BlockSpec((BN, D), lambda i: (i, 0)),
        out_shape=jax.ShapeDtypeStruct((NPAD, D), jnp.float32),
    )(p, p, x, wself, b.reshape(1, D))


def _layer(x, gidx, didx, zeros, v, comp, wself, b, relu):
    table = _make_table(x, v, comp)
    p = _sc_edges_fn()(table, gidx, didx, zeros)
    return _combine(p, x, wself, b, relu)


def kernel(G, emb, etypes, V1, comp1, Wself1, b1, V2, comp2, Wself2, b2):
    src = G[0].astype(jnp.int32)
    dst = G[1].astype(jnp.int32)
    et = etypes.astype(jnp.int32)

    gidx = et * NPAD + src
    npad_e = EPAD - E
    gidx = jnp.concatenate([gidx, jnp.zeros((npad_e,), jnp.int32)])
    didx = jnp.concatenate([dst, jnp.full((npad_e,), N, jnp.int32)])
    gidx = gidx.reshape(NW, K, C)
    didx = didx.reshape(NW, K, C)

    x0 = jnp.pad(emb, ((0, NPAD - N), (0, 0)))
    zeros = jnp.zeros((NPAD, D), jnp.float32)

    h = _layer(x0, gidx, didx, zeros, V1, comp1, Wself1, b1, True)
    h = _layer(h, gidx, didx, zeros, V2, comp2, Wself2, b2, False)
    return h[:N]


# speedup vs baseline: 1.4618x; 1.4618x over previous
import functools

import jax
import jax.numpy as jnp
from jax import lax
from jax.experimental import pallas as pl
from jax.experimental.pallas import tpu as pltpu
from jax.experimental.pallas import tpu_sc as plsc

N = 10000
E = 320000
D = 128
R = 8
B = 4

NPAD = 10240
NC = 2
NS = 16
NW = NC * NS
C = 128
K = -(-E // (NW * C))
EPW = K * C
EPAD = NW * EPW

BN = 2048
NB = NPAD // BN


def _table_body(comp_ref, x_ref, v_ref, out_ref):
    r = pl.program_id(1)
    w = (comp_ref[r, 0] * v_ref[0]
         + comp_ref[r, 1] * v_ref[1]
         + comp_ref[r, 2] * v_ref[2]
         + comp_ref[r, 3] * v_ref[3])
    out_ref[...] = jnp.dot(x_ref[...], w, preferred_element_type=jnp.float32)


def _make_table(x, v, comp):
    return pl.pallas_call(
        _table_body,
        grid=(NB, R),
        in_specs=[
            pl.BlockSpec(memory_space=pltpu.SMEM),
            pl.BlockSpec((BN, D), lambda i, r: (i, 0)),
            pl.BlockSpec((B, D, D), lambda i, r: (0, 0, 0)),
        ],
        out_specs=pl.BlockSpec((BN, D), lambda i, r: (r * NB + i, 0)),
        out_shape=jax.ShapeDtypeStruct((R * NPAD, D), jnp.float32),
    )(comp, x, v)


@functools.cache
def _sc_edges_fn():
    mesh = plsc.VectorSubcoreMesh(
        core_axis_name="c", subcore_axis_name="s",
        num_cores=NC, num_subcores=NS)

    @functools.partial(
        pl.kernel,
        out_type=jax.ShapeDtypeStruct((NC * NPAD, D), jnp.float32),
        mesh=mesh,
        scratch_types=[
            pltpu.VMEM((K, C), jnp.int32),
            pltpu.VMEM((K, C), jnp.int32),
            pltpu.VMEM((C, D), jnp.float32),
            pltpu.VMEM_SHARED((NPAD, D), jnp.float32),
            pltpu.SemaphoreType.DMA,
        ],
    )
    def _sc_edges(table_hbm, gidx_hbm, didx_hbm, zeros_hbm, out_hbm,
                  gidx_v, didx_v, rows0, acc, gs0):
        cid = lax.axis_index("c")
        sid = lax.axis_index("s")
        wid = cid * NS + sid
        stripe = NPAD // NS

        pltpu.sync_copy(zeros_hbm.at[pl.ds(sid * stripe, stripe)],
                        acc.at[pl.ds(sid * stripe, stripe)])
        pltpu.sync_copy(gidx_hbm.at[wid], gidx_v)
        pltpu.sync_copy(didx_hbm.at[wid], didx_v)
        plsc.subcore_barrier()

        def chunk(g, carry):
            pltpu.async_copy(table_hbm.at[gidx_v.at[g]], rows0, gs0).wait()
            pltpu.sync_copy(rows0, acc.at[didx_v.at[g]], add=True)
            return carry

        lax.fori_loop(0, K, chunk, 0)
        plsc.subcore_barrier()

        pltpu.sync_copy(acc.at[pl.ds(sid * stripe, stripe)],
                        out_hbm.at[pl.ds(cid * NPAD + sid * stripe, stripe)])

    return _sc_edges


def _combine_body(p0_ref, p1_ref, x_ref, w_ref, b_ref, out_ref, *, relu):
    h = (p0_ref[...] + p1_ref[...] + b_ref[...]
         + jnp.dot(x_ref[...], w_ref[...], preferred_element_type=jnp.float32))
    out_ref[...] = jnp.maximum(h, 0.0) if relu else h


def _combine(p, x, wself, b, relu):
    return pl.pallas_call(
        functools.partial(_combine_body, relu=relu),
        grid=(NB,),
        in_specs=[
            pl.BlockSpec((BN, D), lambda i: (i, 0)),
            pl.BlockSpec((BN, D), lambda i: (NB + i, 0)),
            pl.BlockSpec((BN, D), lambda i: (i, 0)),
            pl.BlockSpec((D, D), lambda i: (0, 0)),
            pl.BlockSpec((1, D), lambda i: (0, 0)),
        ],
        out_specs=pl.BlockSpec((BN, D), lambda i: (i, 0)),
        out_shape=jax.ShapeDtypeStruct((NPAD, D), jnp.float32),
    )(p, p, x, wself, b.reshape(1, D))


def _layer(x, gidx, didx, zeros, v, comp, wself, b, relu):
    table = _make_table(x, v, comp)
    p = _sc_edges_fn()(table, gidx, didx, zeros)
    return _combine(p, x, wself, b, relu)


def kernel(G, emb, etypes, V1, comp1, Wself1, b1, V2, comp2, Wself2, b2):
    src = G[0].astype(jnp.int32)
    dst = G[1].astype(jnp.int32)
    et = etypes.astype(jnp.int32)

    gidx = et * NPAD + src
    npad_e = EPAD - E
    gidx = jnp.concatenate([gidx, jnp.zeros((npad_e,), jnp.int32)])
    didx = jnp.concatenate([dst, jnp.full((npad_e,), N, jnp.int32)])
    gidx = gidx.reshape(NW, K, C)
    didx = didx.reshape(NW, K, C)

    x0 = jnp.pad(emb, ((0, NPAD - N), (0, 0)))
    zeros = jnp.zeros((NPAD, D), jnp.float32)

    h = _layer(x0, gidx, didx, zeros, V1, comp1, Wself1, b1, True)
    h = _layer(h, gidx, didx, zeros, V2, comp2, Wself2, b2, False)
    return h[:N]


# speedup vs baseline: 2.4430x; 1.6713x over previous
import functools

import jax
import jax.numpy as jnp
from jax import lax
from jax.experimental import pallas as pl
from jax.experimental.pallas import tpu as pltpu
from jax.experimental.pallas import tpu_sc as plsc

N = 10000
E = 320000
D = 128
R = 8
B = 4

NPAD = 10240
NC = 2
NS = 16
NW = NC * NS
C = 128
K = -(-E // (NW * C))
EPW = K * C
EPAD = NW * EPW

BN = 2048
NB = NPAD // BN


def _table_body(comp_ref, x_ref, v_ref, out_ref):
    r = pl.program_id(1)
    w = (comp_ref[r, 0] * v_ref[0]
         + comp_ref[r, 1] * v_ref[1]
         + comp_ref[r, 2] * v_ref[2]
         + comp_ref[r, 3] * v_ref[3])
    out_ref[...] = jnp.dot(x_ref[...], w, preferred_element_type=jnp.float32)


def _make_table(x, v, comp):
    return pl.pallas_call(
        _table_body,
        grid=(NB, R),
        in_specs=[
            pl.BlockSpec(memory_space=pltpu.SMEM),
            pl.BlockSpec((BN, D), lambda i, r: (i, 0)),
            pl.BlockSpec((B, D, D), lambda i, r: (0, 0, 0)),
        ],
        out_specs=pl.BlockSpec((BN, D), lambda i, r: (r * NB + i, 0)),
        out_shape=jax.ShapeDtypeStruct((R * NPAD, D), jnp.float32),
    )(comp, x, v)


@functools.cache
def _sc_edges_fn():
    mesh = plsc.VectorSubcoreMesh(
        core_axis_name="c", subcore_axis_name="s",
        num_cores=NC, num_subcores=NS)

    @functools.partial(
        pl.kernel,
        out_type=jax.ShapeDtypeStruct((NC * NPAD, D), jnp.float32),
        mesh=mesh,
        scratch_types=[
            pltpu.VMEM((K, C), jnp.int32),
            pltpu.VMEM((K, C), jnp.int32),
            pltpu.VMEM((C, D), jnp.float32),
            pltpu.VMEM_SHARED((NPAD, D), jnp.float32),
            pltpu.SemaphoreType.DMA,
        ],
    )
    def _sc_edges(table_hbm, gidx_hbm, didx_hbm, zeros_hbm, out_hbm,
                  gidx_v, didx_v, rows0, acc, gs0):
        cid = lax.axis_index("c")
        sid = lax.axis_index("s")
        wid = cid * NS + sid
        stripe = NPAD // NS

        pltpu.sync_copy(zeros_hbm.at[pl.ds(sid * stripe, stripe)],
                        acc.at[pl.ds(sid * stripe, stripe)])
        pltpu.sync_copy(gidx_hbm.at[wid], gidx_v)
        pltpu.sync_copy(didx_hbm.at[wid], didx_v)
        plsc.subcore_barrier()

        def chunk(g, carry):
            pltpu.async_copy(table_hbm.at[gidx_v.at[g]], rows0, gs0).wait()
            pltpu.sync_copy(rows0, acc.at[didx_v.at[g]], add=True)
            return carry

        lax.fori_loop(0, K, chunk, 0)
        plsc.subcore_barrier()

        pltpu.sync_copy(acc.at[pl.ds(sid * stripe, stripe)],
                        out_hbm.at[pl.ds(cid * NPAD + sid * stripe, stripe)])

    return _sc_edges


def _combine_body(p0_ref, p1_ref, x_ref, w_ref, b_ref, out_ref, *, relu):
    h = (p0_ref[...] + p1_ref[...] + b_ref[...]
         + jnp.dot(x_ref[...], w_ref[...], preferred_element_type=jnp.float32))
    out_ref[...] = jnp.maximum(h, 0.0) if relu else h


def _combine(p, x, wself, b, relu):
    return pl.pallas_call(
        functools.partial(_combine_body, relu=relu),
        grid=(NB,),
        in_specs=[
            pl.BlockSpec((BN, D), lambda i: (i, 0)),
            pl.BlockSpec((BN, D), lambda i: (NB + i, 0)),
            pl.BlockSpec((BN, D), lambda i: (i, 0)),
            pl.BlockSpec((D, D), lambda i: (0, 0)),
            pl.BlockSpec((1, D), lambda i: (0, 0)),
        ],
        out_specs=pl.BlockSpec((BN, D), lambda i: (i, 0)),
        out_shape=jax.ShapeDtypeStruct((NPAD, D), jnp.float32),
    )(p, p, x, wself, b.reshape(1, D))


def _layer(x, gidx, didx, zeros, v, comp, wself, b, relu):
    table = _make_table(x, v, comp)
    p = _sc_edges_fn()(table, gidx, didx, zeros)
    return _combine(p, x, wself, b, relu)


def kernel(G, emb, etypes, V1, comp1, Wself1, b1, V2, comp2, Wself2, b2):
    src = G[0].astype(jnp.int32)
    dst = G[1].astype(jnp.int32)
    et = etypes.astype(jnp.int32)

    gidx = et * NPAD + src
    npad_e = EPAD - E
    pad_g = jnp.arange(npad_e, dtype=jnp.int32) % (R * NPAD)
    pad_d = N + jnp.arange(npad_e, dtype=jnp.int32) % (NPAD - N)
    gidx = jnp.concatenate([gidx, pad_g])
    didx = jnp.concatenate([dst, pad_d])
    gidx = gidx.reshape(NW, K, C)
    didx = didx.reshape(NW, K, C)

    x0 = jnp.pad(emb, ((0, NPAD - N), (0, 0)))
    zeros = jnp.zeros((NPAD, D), jnp.float32)

    h = _layer(x0, gidx, didx, zeros, V1, comp1, Wself1, b1, True)
    h = _layer(h, gidx, didx, zeros, V2, comp2, Wself2, b2, False)
    return h[:N]


# speedup vs baseline: 3.4232x; 1.4012x over previous
import functools

import jax
import jax.numpy as jnp
from jax import lax
from jax.experimental import pallas as pl
from jax.experimental.pallas import tpu as pltpu
from jax.experimental.pallas import tpu_sc as plsc

N = 10000
E = 320000
D = 128
R = 8
B = 4

NPAD = 10240
NC = 2
NS = 16
NW = NC * NS
C = 128
K = 2 * (-(-E // (NW * C * 2)))
EPW = K * C
EPAD = NW * EPW

BN = 2048
NB = NPAD // BN


def _table_body(comp_ref, x_ref, v_ref, out_ref):
    r = pl.program_id(1)
    w = (comp_ref[r, 0] * v_ref[0]
         + comp_ref[r, 1] * v_ref[1]
         + comp_ref[r, 2] * v_ref[2]
         + comp_ref[r, 3] * v_ref[3])
    out_ref[...] = jnp.dot(x_ref[...], w, preferred_element_type=jnp.float32)


def _make_table(x, v, comp):
    return pl.pallas_call(
        _table_body,
        grid=(NB, R),
        in_specs=[
            pl.BlockSpec(memory_space=pltpu.SMEM),
            pl.BlockSpec((BN, D), lambda i, r: (i, 0)),
            pl.BlockSpec((B, D, D), lambda i, r: (0, 0, 0)),
        ],
        out_specs=pl.BlockSpec((BN, D), lambda i, r: (r * NB + i, 0)),
        out_shape=jax.ShapeDtypeStruct((R * NPAD, D), jnp.float32),
    )(comp, x, v)


@functools.cache
def _sc_edges_fn():
    mesh = plsc.VectorSubcoreMesh(
        core_axis_name="c", subcore_axis_name="s",
        num_cores=NC, num_subcores=NS)

    @functools.partial(
        pl.kernel,
        out_type=jax.ShapeDtypeStruct((NC * NPAD, D), jnp.float32),
        mesh=mesh,
        scratch_types=[
            pltpu.VMEM((K, C), jnp.int32),
            pltpu.VMEM((C,), jnp.int32),
            pltpu.VMEM((C,), jnp.int32),
            pltpu.VMEM((C, D), jnp.float32),
            pltpu.VMEM((C, D), jnp.float32),
            pltpu.VMEM_SHARED((NPAD, D), jnp.float32),
            pltpu.SemaphoreType.DMA,
            pltpu.SemaphoreType.DMA,
            pltpu.SemaphoreType.DMA,
            pltpu.SemaphoreType.DMA,
        ],
    )
    def _sc_edges(table_hbm, gidx_hbm, didx_hbm, zeros_hbm, out_hbm,
                  gidx_v, dbuf0, dbuf1, rows0, rows1, acc, gs0, gs1, ds0, ds1):
        cid = lax.axis_index("c")
        sid = lax.axis_index("s")
        wid = cid * NS + sid
        stripe = NPAD // NS

        pltpu.sync_copy(zeros_hbm.at[pl.ds(sid * stripe, stripe)],
                        acc.at[pl.ds(sid * stripe, stripe)])
        pltpu.sync_copy(gidx_hbm.at[wid], gidx_v)
        plsc.subcore_barrier()

        pltpu.async_copy(didx_hbm.at[wid, 0], dbuf0, ds0)
        pltpu.async_copy(didx_hbm.at[wid, 1], dbuf1, ds1)
        pltpu.async_copy(table_hbm.at[gidx_v.at[0]], rows0, gs0)

        def pair(j, carry):
            g = 2 * j
            pltpu.async_copy(table_hbm.at[gidx_v.at[g + 1]], rows1, gs1)
            pltpu.make_async_copy(table_hbm.at[gidx_v.at[g]], rows0, gs0).wait()
            pltpu.make_async_copy(didx_hbm.at[wid, g], dbuf0, ds0).wait()
            pltpu.sync_copy(rows0, acc.at[dbuf0], add=True)

            @pl.when(g + 2 < K)
            def _():
                pltpu.async_copy(didx_hbm.at[wid, g + 2], dbuf0, ds0)
                pltpu.async_copy(table_hbm.at[gidx_v.at[g + 2]], rows0, gs0)

            pltpu.make_async_copy(
                table_hbm.at[gidx_v.at[g + 1]], rows1, gs1).wait()
            pltpu.make_async_copy(didx_hbm.at[wid, g + 1], dbuf1, ds1).wait()
            pltpu.sync_copy(rows1, acc.at[dbuf1], add=True)

            @pl.when(g + 3 < K)
            def _():
                pltpu.async_copy(didx_hbm.at[wid, g + 3], dbuf1, ds1)

            return carry

        lax.fori_loop(0, K // 2, pair, 0)
        plsc.subcore_barrier()

        pltpu.sync_copy(acc.at[pl.ds(sid * stripe, stripe)],
                        out_hbm.at[pl.ds(cid * NPAD + sid * stripe, stripe)])

    return _sc_edges


def _combine_body(p0_ref, p1_ref, x_ref, w_ref, b_ref, out_ref, *, relu):
    h = (p0_ref[...] + p1_ref[...] + b_ref[...]
         + jnp.dot(x_ref[...], w_ref[...], preferred_element_type=jnp.float32))
    out_ref[...] = jnp.maximum(h, 0.0) if relu else h


def _combine(p, x, wself, b, relu):
    return pl.pallas_call(
        functools.partial(_combine_body, relu=relu),
        grid=(NB,),
        in_specs=[
            pl.BlockSpec((BN, D), lambda i: (i, 0)),
            pl.BlockSpec((BN, D), lambda i: (NB + i, 0)),
            pl.BlockSpec((BN, D), lambda i: (i, 0)),
            pl.BlockSpec((D, D), lambda i: (0, 0)),
            pl.BlockSpec((1, D), lambda i: (0, 0)),
        ],
        out_specs=pl.BlockSpec((BN, D), lambda i: (i, 0)),
        out_shape=jax.ShapeDtypeStruct((NPAD, D), jnp.float32),
    )(p, p, x, wself, b.reshape(1, D))


def _layer(x, gidx, didx, zeros, v, comp, wself, b, relu):
    table = _make_table(x, v, comp)
    p = _sc_edges_fn()(table, gidx, didx, zeros)
    return _combine(p, x, wself, b, relu)


def kernel(G, emb, etypes, V1, comp1, Wself1, b1, V2, comp2, Wself2, b2):
    src = G[0].astype(jnp.int32)
    dst = G[1].astype(jnp.int32)
    et = etypes.astype(jnp.int32)

    gidx = et * NPAD + src
    npad_e = EPAD - E
    pad_g = jnp.arange(npad_e, dtype=jnp.int32) % (R * NPAD)
    pad_d = N + jnp.arange(npad_e, dtype=jnp.int32) % (NPAD - N)
    gidx = jnp.concatenate([gidx, pad_g])
    didx = jnp.concatenate([dst, pad_d])
    gidx = gidx.reshape(NW, K, C)
    didx = didx.reshape(NW, K, C)

    x0 = jnp.pad(emb, ((0, NPAD - N), (0, 0)))
    zeros = jnp.zeros((NPAD, D), jnp.float32)

    h = _layer(x0, gidx, didx, zeros, V1, comp1, Wself1, b1, True)
    h = _layer(h, gidx, didx, zeros, V2, comp2, Wself2, b2, False)
    return h[:N]


# speedup vs baseline: 3.6514x; 1.0666x over previous
import functools

import jax
import jax.numpy as jnp
from jax import lax
from jax.experimental import pallas as pl
from jax.experimental.pallas import tpu as pltpu
from jax.experimental.pallas import tpu_sc as plsc

N = 10000
E = 320000
D = 128
R = 8
B = 4

NPAD = 10240
NC = 2
NS = 16
NW = NC * NS
C = 128
K = 2 * (-(-E // (NW * C * 2)))
EPW = K * C
EPAD = NW * EPW

BN = 2048
NB = NPAD // BN


def _table_body(comp_ref, x_ref, v_ref, out_ref):
    r = pl.program_id(1)
    w = (comp_ref[r, 0] * v_ref[0]
         + comp_ref[r, 1] * v_ref[1]
         + comp_ref[r, 2] * v_ref[2]
         + comp_ref[r, 3] * v_ref[3])
    out_ref[...] = jnp.dot(x_ref[...], w, preferred_element_type=jnp.float32)


def _make_table(x, v, comp):
    return pl.pallas_call(
        _table_body,
        grid=(NB, R),
        in_specs=[
            pl.BlockSpec(memory_space=pltpu.SMEM),
            pl.BlockSpec((BN, D), lambda i, r: (i, 0)),
            pl.BlockSpec((B, D, D), lambda i, r: (0, 0, 0)),
        ],
        out_specs=pl.BlockSpec((BN, D), lambda i, r: (r * NB + i, 0)),
        out_shape=jax.ShapeDtypeStruct((R * NPAD, D), jnp.float32),
    )(comp, x, v)


@functools.cache
def _sc_edges_fn():
    mesh = plsc.VectorSubcoreMesh(
        core_axis_name="c", subcore_axis_name="s",
        num_cores=NC, num_subcores=NS)

    @functools.partial(
        pl.kernel,
        out_type=jax.ShapeDtypeStruct((NC * NPAD, D), jnp.float32),
        mesh=mesh,
        scratch_types=[
            pltpu.VMEM((K, C), jnp.int32),
            pltpu.VMEM((C,), jnp.int32),
            pltpu.VMEM((C,), jnp.int32),
            pltpu.VMEM((C, D), jnp.float32),
            pltpu.VMEM((C, D), jnp.float32),
            pltpu.VMEM_SHARED((NPAD, D), jnp.float32),
            pltpu.SemaphoreType.DMA,
            pltpu.SemaphoreType.DMA,
            pltpu.SemaphoreType.DMA,
            pltpu.SemaphoreType.DMA,
        ],
    )
    def _sc_edges(table_hbm, gidx_hbm, didx_hbm, zeros_hbm, out_hbm,
                  gidx_v, dbuf0, dbuf1, rows0, rows1, acc, gs0, gs1, ds0, ds1):
        cid = lax.axis_index("c")
        sid = lax.axis_index("s")
        wid = cid * NS + sid
        stripe = NPAD // NS

        pltpu.sync_copy(zeros_hbm.at[pl.ds(sid * stripe, stripe)],
                        acc.at[pl.ds(sid * stripe, stripe)])
        pltpu.sync_copy(gidx_hbm.at[wid], gidx_v)
        plsc.subcore_barrier()

        pltpu.async_copy(didx_hbm.at[wid, 0], dbuf0, ds0)
        pltpu.async_copy(didx_hbm.at[wid, 1], dbuf1, ds1)
        pltpu.async_copy(table_hbm.at[gidx_v.at[0]], rows0, gs0)

        def pair(j, carry):
            g = 2 * j
            pltpu.async_copy(table_hbm.at[gidx_v.at[g + 1]], rows1, gs1)
            pltpu.make_async_copy(table_hbm.at[gidx_v.at[g]], rows0, gs0).wait()
            pltpu.make_async_copy(didx_hbm.at[wid, g], dbuf0, ds0).wait()
            pltpu.sync_copy(rows0, acc.at[dbuf0], add=True)

            @pl.when(g + 2 < K)
            def _():
                pltpu.async_copy(didx_hbm.at[wid, g + 2], dbuf0, ds0)
                pltpu.async_copy(table_hbm.at[gidx_v.at[g + 2]], rows0, gs0)

            pltpu.make_async_copy(
                table_hbm.at[gidx_v.at[g + 1]], rows1, gs1).wait()
            pltpu.make_async_copy(didx_hbm.at[wid, g + 1], dbuf1, ds1).wait()
            pltpu.sync_copy(rows1, acc.at[dbuf1], add=True)

            @pl.when(g + 3 < K)
            def _():
                pltpu.async_copy(didx_hbm.at[wid, g + 3], dbuf1, ds1)

            return carry

        lax.fori_loop(0, K // 2, pair, 0)
        plsc.subcore_barrier()

        pltpu.sync_copy(acc.at[pl.ds(sid * stripe, stripe)],
                        out_hbm.at[pl.ds(cid * NPAD + sid * stripe, stripe)])

    return _sc_edges


def _combine_body(p0_ref, p1_ref, x_ref, w_ref, b_ref, out_ref, *, relu):
    h = (p0_ref[...] + p1_ref[...] + b_ref[...]
         + jnp.dot(x_ref[...], w_ref[...], preferred_element_type=jnp.float32))
    out_ref[...] = jnp.maximum(h, 0.0) if relu else h


def _comb_table_body(comp_ref, p0_ref, p1_ref, x_ref, w_ref, b_ref, v_ref,
                     t_ref, h_ref):
    h = (p0_ref[...] + p1_ref[...] + b_ref[...]
         + jnp.dot(x_ref[...], w_ref[...], preferred_element_type=jnp.float32))
    h = jnp.maximum(h, 0.0)
    h_ref[...] = h
    for r in range(R):
        w = (comp_ref[r, 0] * v_ref[0]
             + comp_ref[r, 1] * v_ref[1]
             + comp_ref[r, 2] * v_ref[2]
             + comp_ref[r, 3] * v_ref[3])
        t_ref[r] = jnp.dot(h, w, preferred_element_type=jnp.float32)


def _combine_and_table(p, x, wself, b, v, comp):
    t, h = pl.pallas_call(
        _comb_table_body,
        grid=(NB,),
        in_specs=[
            pl.BlockSpec(memory_space=pltpu.SMEM),
            pl.BlockSpec((BN, D), lambda i: (i, 0)),
            pl.BlockSpec((BN, D), lambda i: (NB + i, 0)),
            pl.BlockSpec((BN, D), lambda i: (i, 0)),
            pl.BlockSpec((D, D), lambda i: (0, 0)),
            pl.BlockSpec((1, D), lambda i: (0, 0)),
            pl.BlockSpec((B, D, D), lambda i: (0, 0, 0)),
        ],
        out_specs=[
            pl.BlockSpec((R, BN, D), lambda i: (0, i, 0)),
            pl.BlockSpec((BN, D), lambda i: (i, 0)),
        ],
        out_shape=[
            jax.ShapeDtypeStruct((R, NPAD, D), jnp.float32),
            jax.ShapeDtypeStruct((NPAD, D), jnp.float32),
        ],
    )(comp, p, p, x, wself, b.reshape(1, D), v)
    return t.reshape(R * NPAD, D), h


def _combine(p, x, wself, b, relu):
    return pl.pallas_call(
        functools.partial(_combine_body, relu=relu),
        grid=(NB,),
        in_specs=[
            pl.BlockSpec((BN, D), lambda i: (i, 0)),
            pl.BlockSpec((BN, D), lambda i: (NB + i, 0)),
            pl.BlockSpec((BN, D), lambda i: (i, 0)),
            pl.BlockSpec((D, D), lambda i: (0, 0)),
            pl.BlockSpec((1, D), lambda i: (0, 0)),
        ],
        out_specs=pl.BlockSpec((BN, D), lambda i: (i, 0)),
        out_shape=jax.ShapeDtypeStruct((NPAD, D), jnp.float32),
    )(p, p, x, wself, b.reshape(1, D))


def kernel(G, emb, etypes, V1, comp1, Wself1, b1, V2, comp2, Wself2, b2):
    src = G[0].astype(jnp.int32)
    dst = G[1].astype(jnp.int32)
    et = etypes.astype(jnp.int32)

    gidx = et * NPAD + src
    npad_e = EPAD - E
    pad_g = jnp.arange(npad_e, dtype=jnp.int32) % (R * NPAD)
    pad_d = N + jnp.arange(npad_e, dtype=jnp.int32) % (NPAD - N)
    gidx = jnp.concatenate([gidx, pad_g])
    didx = jnp.concatenate([dst, pad_d])
    gidx = gidx.reshape(NW, K, C)
    didx = didx.reshape(NW, K, C)

    x0 = jnp.pad(emb, ((0, NPAD - N), (0, 0)))
    zeros = jnp.zeros((NPAD, D), jnp.float32)

    sc = _sc_edges_fn()
    table1 = _make_table(x0, V1, comp1)
    p1 = sc(table1, gidx, didx, zeros)
    table2, h1 = _combine_and_table(p1, x0, Wself1, b1, V2, comp2)
    p2 = sc(table2, gidx, didx, zeros)
    h2 = _combine(p2, h1, Wself2, b2, False)
    return h2[:N]


# speedup vs baseline: 3.8620x; 1.0577x over previous
import functools

import jax
import jax.numpy as jnp
from jax import lax
from jax.experimental import pallas as pl
from jax.experimental.pallas import tpu as pltpu
from jax.experimental.pallas import tpu_sc as plsc

N = 10000
E = 320000
D = 128
R = 8
B = 4

NPAD = 10240
NC = 2
NS = 16
NW = NC * NS
C = 96
K = 6 * (-(-E // (NW * C * 6)))
EPW = K * C
EPAD = NW * EPW
NACC = 10112

BN = 2048
NB = NPAD // BN


def _table_body(comp_ref, x_ref, v_ref, out_ref):
    r = pl.program_id(1)
    w = (comp_ref[r, 0] * v_ref[0]
         + comp_ref[r, 1] * v_ref[1]
         + comp_ref[r, 2] * v_ref[2]
         + comp_ref[r, 3] * v_ref[3])
    out_ref[...] = jnp.dot(x_ref[...], w, preferred_element_type=jnp.float32)


def _make_table(x, v, comp):
    return pl.pallas_call(
        _table_body,
        grid=(NB, R),
        in_specs=[
            pl.BlockSpec(memory_space=pltpu.SMEM),
            pl.BlockSpec((BN, D), lambda i, r: (i, 0)),
            pl.BlockSpec((B, D, D), lambda i, r: (0, 0, 0)),
        ],
        out_specs=pl.BlockSpec((BN, D), lambda i, r: (r * NB + i, 0)),
        out_shape=jax.ShapeDtypeStruct((R * NPAD, D), jnp.float32),
    )(comp, x, v)


@functools.cache
def _sc_edges_fn():
    mesh = plsc.VectorSubcoreMesh(
        core_axis_name="c", subcore_axis_name="s",
        num_cores=NC, num_subcores=NS)

    @functools.partial(
        pl.kernel,
        out_type=jax.ShapeDtypeStruct((NC * NPAD, D), jnp.float32),
        mesh=mesh,
        scratch_types=[
            [pltpu.VMEM((C, D), jnp.float32) for _ in range(3)],
            [[pltpu.VMEM((C,), jnp.int32) for _ in range(2)] for _ in range(3)],
            [[pltpu.VMEM((C,), jnp.int32) for _ in range(2)] for _ in range(3)],
            pltpu.VMEM_SHARED((NACC, D), jnp.float32),
            [pltpu.SemaphoreType.DMA for _ in range(3)],
            [[pltpu.SemaphoreType.DMA for _ in range(2)] for _ in range(3)],
            [[pltpu.SemaphoreType.DMA for _ in range(2)] for _ in range(3)],
        ],
    )
    def _sc_edges(table_hbm, gidx_hbm, didx_hbm, zeros_hbm, out_hbm,
                  rows, gbuf, dbuf, acc, gs, isem, dsem):
        cid = lax.axis_index("c")
        sid = lax.axis_index("s")
        wid = cid * NS + sid
        stripe = NACC // NS

        pltpu.sync_copy(zeros_hbm.at[pl.ds(sid * stripe, stripe)],
                        acc.at[pl.ds(sid * stripe, stripe)])
        plsc.subcore_barrier()

        for p in range(2):
            for u in range(3):
                c = 3 * p + u
                pltpu.async_copy(gidx_hbm.at[wid, c], gbuf[u][p], isem[u][p])
                pltpu.async_copy(didx_hbm.at[wid, c], dbuf[u][p], dsem[u][p])
        for u in range(3):
            pltpu.make_async_copy(
                gidx_hbm.at[wid, u], gbuf[u][0], isem[u][0]).wait()
            pltpu.async_copy(table_hbm.at[gbuf[u][0]], rows[u], gs[u])

        def slot(t, u, p, q):
            c = 3 * t + u
            pltpu.make_async_copy(
                table_hbm.at[gbuf[u][p]], rows[u], gs[u]).wait()
            pltpu.make_async_copy(
                didx_hbm.at[wid, c], dbuf[u][p], dsem[u][p]).wait()
            pltpu.sync_copy(rows[u], acc.at[dbuf[u][p]], add=True)
            nc1 = c + 3

            @pl.when(nc1 < K)
            def _():
                pltpu.make_async_copy(
                    gidx_hbm.at[wid, nc1], gbuf[u][q], isem[u][q]).wait()
                pltpu.async_copy(table_hbm.at[gbuf[u][q]], rows[u], gs[u])

            nc2 = c + 6

            @pl.when(nc2 < K)
            def _():
                pltpu.async_copy(gidx_hbm.at[wid, nc2], gbuf[u][p], isem[u][p])
                pltpu.async_copy(didx_hbm.at[wid, nc2], dbuf[u][p], dsem[u][p])

        def pair(m, carry):
            t = 2 * m
            for u in range(3):
                slot(t, u, 0, 1)
            for u in range(3):
                slot(t + 1, u, 1, 0)
            return carry

        lax.fori_loop(0, K // 6, pair, 0)
        plsc.subcore_barrier()

        pltpu.sync_copy(acc.at[pl.ds(sid * stripe, stripe)],
                        out_hbm.at[pl.ds(cid * NPAD + sid * stripe, stripe)])

    return _sc_edges


def _combine_body(p0_ref, p1_ref, x_ref, w_ref, b_ref, out_ref, *, relu):
    h = (p0_ref[...] + p1_ref[...] + b_ref[...]
         + jnp.dot(x_ref[...], w_ref[...], preferred_element_type=jnp.float32))
    out_ref[...] = jnp.maximum(h, 0.0) if relu else h


def _comb_table_body(comp_ref, p0_ref, p1_ref, x_ref, w_ref, b_ref, v_ref,
                     t_ref, h_ref):
    h = (p0_ref[...] + p1_ref[...] + b_ref[...]
         + jnp.dot(x_ref[...], w_ref[...], preferred_element_type=jnp.float32))
    h = jnp.maximum(h, 0.0)
    h_ref[...] = h
    for r in range(R):
        w = (comp_ref[r, 0] * v_ref[0]
             + comp_ref[r, 1] * v_ref[1]
             + comp_ref[r, 2] * v_ref[2]
             + comp_ref[r, 3] * v_ref[3])
        t_ref[r] = jnp.dot(h, w, preferred_element_type=jnp.float32)


def _combine_and_table(p, x, wself, b, v, comp):
    t, h = pl.pallas_call(
        _comb_table_body,
        grid=(NB,),
        in_specs=[
            pl.BlockSpec(memory_space=pltpu.SMEM),
            pl.BlockSpec((BN, D), lambda i: (i, 0)),
            pl.BlockSpec((BN, D), lambda i: (NB + i, 0)),
            pl.BlockSpec((BN, D), lambda i: (i, 0)),
            pl.BlockSpec((D, D), lambda i: (0, 0)),
            pl.BlockSpec((1, D), lambda i: (0, 0)),
            pl.BlockSpec((B, D, D), lambda i: (0, 0, 0)),
        ],
        out_specs=[
            pl.BlockSpec((R, BN, D), lambda i: (0, i, 0)),
            pl.BlockSpec((BN, D), lambda i: (i, 0)),
        ],
        out_shape=[
            jax.ShapeDtypeStruct((R, NPAD, D), jnp.float32),
            jax.ShapeDtypeStruct((NPAD, D), jnp.float32),
        ],
    )(comp, p, p, x, wself, b.reshape(1, D), v)
    return t.reshape(R * NPAD, D), h


def _combine(p, x, wself, b, relu):
    return pl.pallas_call(
        functools.partial(_combine_body, relu=relu),
        grid=(NB,),
        in_specs=[
            pl.BlockSpec((BN, D), lambda i: (i, 0)),
            pl.BlockSpec((BN, D), lambda i: (NB + i, 0)),
            pl.BlockSpec((BN, D), lambda i: (i, 0)),
            pl.BlockSpec((D, D), lambda i: (0, 0)),
            pl.BlockSpec((1, D), lambda i: (0, 0)),
        ],
        out_specs=pl.BlockSpec((BN, D), lambda i: (i, 0)),
        out_shape=jax.ShapeDtypeStruct((NPAD, D), jnp.float32),
    )(p, p, x, wself, b.reshape(1, D))


def kernel(G, emb, etypes, V1, comp1, Wself1, b1, V2, comp2, Wself2, b2):
    src = G[0].astype(jnp.int32)
    dst = G[1].astype(jnp.int32)
    et = etypes.astype(jnp.int32)

    gidx = et * NPAD + src
    npad_e = EPAD - E
    pad_g = jnp.arange(npad_e, dtype=jnp.int32) % (R * NPAD)
    pad_d = N + jnp.arange(npad_e, dtype=jnp.int32) % (NACC - N)
    gidx = jnp.concatenate([gidx, pad_g])
    didx = jnp.concatenate([dst, pad_d])
    gidx = gidx.reshape(NW, K, C)
    didx = didx.reshape(NW, K, C)

    x0 = jnp.pad(emb, ((0, NPAD - N), (0, 0)))
    zeros = jnp.zeros((NPAD, D), jnp.float32)

    sc = _sc_edges_fn()
    table1 = _make_table(x0, V1, comp1)
    p1 = sc(table1, gidx, didx, zeros)
    table2, h1 = _combine_and_table(p1, x0, Wself1, b1, V2, comp2)
    p2 = sc(table2, gidx, didx, zeros)
    h2 = _combine(p2, h1, Wself2, b2, False)
    return h2[:N]


# speedup vs baseline: 3.8646x; 1.0007x over previous
import functools

import jax
import jax.numpy as jnp
from jax import lax
from jax.experimental import pallas as pl
from jax.experimental.pallas import tpu as pltpu
from jax.experimental.pallas import tpu_sc as plsc

N = 10000
E = 320000
D = 128
R = 8
B = 4

NPAD = 10240
NC = 2
NS = 16
NW = NC * NS
C = 96
K = 6 * (-(-E // (NW * C * 6)))
EPW = K * C
EPAD = NW * EPW
NACC = 10112

BN = 2048
NB = NPAD // BN


def _table_body(comp_ref, x_ref, v_ref, out_ref):
    r = pl.program_id(1)
    w = (comp_ref[r, 0] * v_ref[0]
         + comp_ref[r, 1] * v_ref[1]
         + comp_ref[r, 2] * v_ref[2]
         + comp_ref[r, 3] * v_ref[3])
    out_ref[...] = jnp.dot(x_ref[...].astype(jnp.bfloat16),
                           w.astype(jnp.bfloat16),
                           preferred_element_type=jnp.float32)


def _make_table(x, v, comp):
    return pl.pallas_call(
        _table_body,
        grid=(NB, R),
        in_specs=[
            pl.BlockSpec(memory_space=pltpu.SMEM),
            pl.BlockSpec((BN, D), lambda i, r: (i, 0)),
            pl.BlockSpec((B, D, D), lambda i, r: (0, 0, 0)),
        ],
        out_specs=pl.BlockSpec((BN, D), lambda i, r: (r * NB + i, 0)),
        out_shape=jax.ShapeDtypeStruct((R * NPAD, D), jnp.float32),
    )(comp, x, v)


@functools.cache
def _sc_edges_fn():
    mesh = plsc.VectorSubcoreMesh(
        core_axis_name="c", subcore_axis_name="s",
        num_cores=NC, num_subcores=NS)

    @functools.partial(
        pl.kernel,
        out_type=jax.ShapeDtypeStruct((NC * NPAD, D), jnp.float32),
        mesh=mesh,
        scratch_types=[
            [pltpu.VMEM((C, D), jnp.float32) for _ in range(3)],
            [[pltpu.VMEM((C,), jnp.int32) for _ in range(2)] for _ in range(3)],
            [[pltpu.VMEM((C,), jnp.int32) for _ in range(2)] for _ in range(3)],
            pltpu.VMEM_SHARED((NACC, D), jnp.float32),
            [pltpu.SemaphoreType.DMA for _ in range(3)],
            [[pltpu.SemaphoreType.DMA for _ in range(2)] for _ in range(3)],
            [[pltpu.SemaphoreType.DMA for _ in range(2)] for _ in range(3)],
        ],
    )
    def _sc_edges(table_hbm, gidx_hbm, didx_hbm, zeros_hbm, out_hbm,
                  rows, gbuf, dbuf, acc, gs, isem, dsem):
        cid = lax.axis_index("c")
        sid = lax.axis_index("s")
        wid = cid * NS + sid
        stripe = NACC // NS

        pltpu.sync_copy(zeros_hbm.at[pl.ds(sid * stripe, stripe)],
                        acc.at[pl.ds(sid * stripe, stripe)])
        plsc.subcore_barrier()

        for p in range(2):
            for u in range(3):
                c = 3 * p + u
                pltpu.async_copy(gidx_hbm.at[wid, c], gbuf[u][p], isem[u][p])
                pltpu.async_copy(didx_hbm.at[wid, c], dbuf[u][p], dsem[u][p])
        for u in range(3):
            pltpu.make_async_copy(
                gidx_hbm.at[wid, u], gbuf[u][0], isem[u][0]).wait()
            pltpu.async_copy(table_hbm.at[gbuf[u][0]], rows[u], gs[u])

        def slot(t, u, p, q):
            c = 3 * t + u
            pltpu.make_async_copy(
                table_hbm.at[gbuf[u][p]], rows[u], gs[u]).wait()
            pltpu.make_async_copy(
                didx_hbm.at[wid, c], dbuf[u][p], dsem[u][p]).wait()
            pltpu.sync_copy(rows[u], acc.at[dbuf[u][p]], add=True)
            nc1 = c + 3

            @pl.when(nc1 < K)
            def _():
                pltpu.make_async_copy(
                    gidx_hbm.at[wid, nc1], gbuf[u][q], isem[u][q]).wait()
                pltpu.async_copy(table_hbm.at[gbuf[u][q]], rows[u], gs[u])

            nc2 = c + 6

            @pl.when(nc2 < K)
            def _():
                pltpu.async_copy(gidx_hbm.at[wid, nc2], gbuf[u][p], isem[u][p])
                pltpu.async_copy(didx_hbm.at[wid, nc2], dbuf[u][p], dsem[u][p])

        def pair(m, carry):
            t = 2 * m
            for u in range(3):
                slot(t, u, 0, 1)
            for u in range(3):
                slot(t + 1, u, 1, 0)
            return carry

        lax.fori_loop(0, K // 6, pair, 0)
        plsc.subcore_barrier()

        pltpu.sync_copy(acc.at[pl.ds(sid * stripe, stripe)],
                        out_hbm.at[pl.ds(cid * NPAD + sid * stripe, stripe)])

    return _sc_edges


def _combine_body(p0_ref, p1_ref, x_ref, w_ref, b_ref, out_ref, *, relu):
    h = (p0_ref[...] + p1_ref[...] + b_ref[...]
         + jnp.dot(x_ref[...].astype(jnp.bfloat16),
                   w_ref[...].astype(jnp.bfloat16),
                   preferred_element_type=jnp.float32))
    out_ref[...] = jnp.maximum(h, 0.0) if relu else h


def _comb_table_body(comp_ref, p0_ref, p1_ref, x_ref, w_ref, b_ref, v_ref,
                     t_ref, h_ref):
    h = (p0_ref[...] + p1_ref[...] + b_ref[...]
         + jnp.dot(x_ref[...].astype(jnp.bfloat16),
                   w_ref[...].astype(jnp.bfloat16),
                   preferred_element_type=jnp.float32))
    h = jnp.maximum(h, 0.0)
    h_ref[...] = h
    hb = h.astype(jnp.bfloat16)
    for r in range(R):
        w = (comp_ref[r, 0] * v_ref[0]
             + comp_ref[r, 1] * v_ref[1]
             + comp_ref[r, 2] * v_ref[2]
             + comp_ref[r, 3] * v_ref[3])
        t_ref[r] = jnp.dot(hb, w.astype(jnp.bfloat16),
                           preferred_element_type=jnp.float32)


def _combine_and_table(p, x, wself, b, v, comp):
    t, h = pl.pallas_call(
        _comb_table_body,
        grid=(NB,),
        in_specs=[
            pl.BlockSpec(memory_space=pltpu.SMEM),
            pl.BlockSpec((BN, D), lambda i: (i, 0)),
            pl.BlockSpec((BN, D), lambda i: (NB + i, 0)),
            pl.BlockSpec((BN, D), lambda i: (i, 0)),
            pl.BlockSpec((D, D), lambda i: (0, 0)),
            pl.BlockSpec((1, D), lambda i: (0, 0)),
            pl.BlockSpec((B, D, D), lambda i: (0, 0, 0)),
        ],
        out_specs=[
            pl.BlockSpec((R, BN, D), lambda i: (0, i, 0)),
            pl.BlockSpec((BN, D), lambda i: (i, 0)),
        ],
        out_shape=[
            jax.ShapeDtypeStruct((R, NPAD, D), jnp.float32),
            jax.ShapeDtypeStruct((NPAD, D), jnp.float32),
        ],
    )(comp, p, p, x, wself, b.reshape(1, D), v)
    return t.reshape(R * NPAD, D), h


def _combine(p, x, wself, b, relu):
    return pl.pallas_call(
        functools.partial(_combine_body, relu=relu),
        grid=(NB,),
        in_specs=[
            pl.BlockSpec((BN, D), lambda i: (i, 0)),
            pl.BlockSpec((BN, D), lambda i: (NB + i, 0)),
            pl.BlockSpec((BN, D), lambda i: (i, 0)),
            pl.BlockSpec((D, D), lambda i: (0, 0)),
            pl.BlockSpec((1, D), lambda i: (0, 0)),
        ],
        out_specs=pl.BlockSpec((BN, D), lambda i: (i, 0)),
        out_shape=jax.ShapeDtypeStruct((NPAD, D), jnp.float32),
    )(p, p, x, wself, b.reshape(1, D))


def kernel(G, emb, etypes, V1, comp1, Wself1, b1, V2, comp2, Wself2, b2):
    src = G[0].astype(jnp.int32)
    dst = G[1].astype(jnp.int32)
    et = etypes.astype(jnp.int32)

    gidx = et * NPAD + src
    npad_e = EPAD - E
    pad_g = jnp.arange(npad_e, dtype=jnp.int32) % (R * NPAD)
    pad_d = N + jnp.arange(npad_e, dtype=jnp.int32) % (NACC - N)
    gidx = jnp.concatenate([gidx, pad_g])
    didx = jnp.concatenate([dst, pad_d])
    gidx = gidx.reshape(NW, K, C)
    didx = didx.reshape(NW, K, C)

    x0 = jnp.pad(emb, ((0, NPAD - N), (0, 0)))
    zeros = jnp.zeros((NPAD, D), jnp.float32)

    sc = _sc_edges_fn()
    table1 = _make_table(x0, V1, comp1)
    p1 = sc(table1, gidx, didx, zeros)
    table2, h1 = _combine_and_table(p1, x0, Wself1, b1, V2, comp2)
    p2 = sc(table2, gidx, didx, zeros)
    h2 = _combine(p2, h1, Wself2, b2, False)
    return h2[:N]


# speedup vs baseline: 3.8660x; 1.0003x over previous
import functools

import jax
import jax.numpy as jnp
from jax import lax
from jax.experimental import pallas as pl
from jax.experimental.pallas import tpu as pltpu
from jax.experimental.pallas import tpu_sc as plsc

N = 10000
E = 320000
D = 128
R = 8
B = 4

NPAD = 10240
NC = 2
NS = 16
NW = NC * NS
C = 96
K = 6 * (-(-E // (NW * C * 6)))
EPW = K * C
EPAD = NW * EPW
NACC = 10112

BN = 2048
NB = NPAD // BN


def _table_body(comp_ref, x_ref, v_ref, out_ref):
    r = pl.program_id(1)
    w = (comp_ref[r, 0] * v_ref[0]
         + comp_ref[r, 1] * v_ref[1]
         + comp_ref[r, 2] * v_ref[2]
         + comp_ref[r, 3] * v_ref[3])
    out_ref[...] = jnp.dot(x_ref[...], w, preferred_element_type=jnp.float32)


def _make_table(x, v, comp):
    return pl.pallas_call(
        _table_body,
        grid=(NB, R),
        in_specs=[
            pl.BlockSpec(memory_space=pltpu.SMEM),
            pl.BlockSpec((BN, D), lambda i, r: (i, 0)),
            pl.BlockSpec((B, D, D), lambda i, r: (0, 0, 0)),
        ],
        out_specs=pl.BlockSpec((BN, D), lambda i, r: (r * NB + i, 0)),
        out_shape=jax.ShapeDtypeStruct((R * NPAD, D), jnp.float32),
    )(comp, x, v)


@functools.cache
def _sc_edges_fn():
    mesh = plsc.VectorSubcoreMesh(
        core_axis_name="c", subcore_axis_name="s",
        num_cores=NC, num_subcores=NS)

    @functools.partial(
        pl.kernel,
        out_type=jax.ShapeDtypeStruct((NC * NPAD, D), jnp.float32),
        mesh=mesh,
        scratch_types=[
            [pltpu.VMEM((C, D), jnp.float32) for _ in range(3)],
            [[pltpu.VMEM((C,), jnp.int32) for _ in range(2)] for _ in range(3)],
            [[pltpu.VMEM((C,), jnp.int32) for _ in range(2)] for _ in range(3)],
            pltpu.VMEM_SHARED((NACC, D), jnp.float32),
            [pltpu.SemaphoreType.DMA for _ in range(3)],
            [[pltpu.SemaphoreType.DMA for _ in range(2)] for _ in range(3)],
            [[pltpu.SemaphoreType.DMA for _ in range(2)] for _ in range(3)],
        ],
    )
    def _sc_edges(table_hbm, gidx_hbm, didx_hbm, zeros_hbm, out_hbm,
                  rows, gbuf, dbuf, acc, gs, isem, dsem):
        cid = lax.axis_index("c")
        sid = lax.axis_index("s")
        wid = cid * NS + sid
        stripe = NACC // NS

        pltpu.sync_copy(zeros_hbm.at[pl.ds(sid * stripe, stripe)],
                        acc.at[pl.ds(sid * stripe, stripe)])
        plsc.subcore_barrier()

        for p in range(2):
            for u in range(3):
                c = 3 * p + u
                pltpu.async_copy(gidx_hbm.at[wid, c], gbuf[u][p], isem[u][p])
                pltpu.async_copy(didx_hbm.at[wid, c], dbuf[u][p], dsem[u][p])
        for u in range(3):
            pltpu.make_async_copy(
                gidx_hbm.at[wid, u], gbuf[u][0], isem[u][0]).wait()
            pltpu.async_copy(table_hbm.at[gbuf[u][0]], rows[u], gs[u])

        def slot(t, u, p, q):
            c = 3 * t + u
            pltpu.make_async_copy(
                table_hbm.at[gbuf[u][p]], rows[u], gs[u]).wait()
            pltpu.make_async_copy(
                didx_hbm.at[wid, c], dbuf[u][p], dsem[u][p]).wait()
            pltpu.sync_copy(rows[u], acc.at[dbuf[u][p]], add=True)
            nc1 = c + 3

            @pl.when(nc1 < K)
            def _():
                pltpu.make_async_copy(
                    gidx_hbm.at[wid, nc1], gbuf[u][q], isem[u][q]).wait()
                pltpu.async_copy(table_hbm.at[gbuf[u][q]], rows[u], gs[u])

            nc2 = c + 6

            @pl.when(nc2 < K)
            def _():
                pltpu.async_copy(gidx_hbm.at[wid, nc2], gbuf[u][p], isem[u][p])
                pltpu.async_copy(didx_hbm.at[wid, nc2], dbuf[u][p], dsem[u][p])

        def pair(m, carry):
            t = 2 * m
            for u in range(3):
                slot(t, u, 0, 1)
            for u in range(3):
                slot(t + 1, u, 1, 0)
            return carry

        lax.fori_loop(0, K // 6, pair, 0)
        plsc.subcore_barrier()

        pltpu.sync_copy(acc.at[pl.ds(sid * stripe, stripe)],
                        out_hbm.at[pl.ds(cid * NPAD + sid * stripe, stripe)])

    return _sc_edges


def _combine_body(p0_ref, p1_ref, x_ref, w_ref, b_ref, out_ref, *, relu):
    h = (p0_ref[...] + p1_ref[...] + b_ref[...]
         + jnp.dot(x_ref[...], w_ref[...], preferred_element_type=jnp.float32))
    out_ref[...] = jnp.maximum(h, 0.0) if relu else h


def _comb_table_body(comp_ref, p0_ref, p1_ref, x_ref, w_ref, b_ref, v_ref,
                     t_ref, h_ref):
    h = (p0_ref[...] + p1_ref[...] + b_ref[...]
         + jnp.dot(x_ref[...], w_ref[...], preferred_element_type=jnp.float32))
    h = jnp.maximum(h, 0.0)
    h_ref[...] = h
    for r in range(R):
        w = (comp_ref[r, 0] * v_ref[0]
             + comp_ref[r, 1] * v_ref[1]
             + comp_ref[r, 2] * v_ref[2]
             + comp_ref[r, 3] * v_ref[3])
        t_ref[r] = jnp.dot(h, w, preferred_element_type=jnp.float32)


def _combine_and_table(p, x, wself, b, v, comp):
    t, h = pl.pallas_call(
        _comb_table_body,
        grid=(NB,),
        in_specs=[
            pl.BlockSpec(memory_space=pltpu.SMEM),
            pl.BlockSpec((BN, D), lambda i: (i, 0)),
            pl.BlockSpec((BN, D), lambda i: (NB + i, 0)),
            pl.BlockSpec((BN, D), lambda i: (i, 0)),
            pl.BlockSpec((D, D), lambda i: (0, 0)),
            pl.BlockSpec((1, D), lambda i: (0, 0)),
            pl.BlockSpec((B, D, D), lambda i: (0, 0, 0)),
        ],
        out_specs=[
            pl.BlockSpec((R, BN, D), lambda i: (0, i, 0)),
            pl.BlockSpec((BN, D), lambda i: (i, 0)),
        ],
        out_shape=[
            jax.ShapeDtypeStruct((R, NPAD, D), jnp.float32),
            jax.ShapeDtypeStruct((NPAD, D), jnp.float32),
        ],
    )(comp, p, p, x, wself, b.reshape(1, D), v)
    return t.reshape(R * NPAD, D), h


def _combine(p, x, wself, b, relu):
    return pl.pallas_call(
        functools.partial(_combine_body, relu=relu),
        grid=(NB,),
        in_specs=[
            pl.BlockSpec((BN, D), lambda i: (i, 0)),
            pl.BlockSpec((BN, D), lambda i: (NB + i, 0)),
            pl.BlockSpec((BN, D), lambda i: (i, 0)),
            pl.BlockSpec((D, D), lambda i: (0, 0)),
            pl.BlockSpec((1, D), lambda i: (0, 0)),
        ],
        out_specs=pl.BlockSpec((BN, D), lambda i: (i, 0)),
        out_shape=jax.ShapeDtypeStruct((NPAD, D), jnp.float32),
    )(p, p, x, wself, b.reshape(1, D))


def kernel(G, emb, etypes, V1, comp1, Wself1, b1, V2, comp2, Wself2, b2):
    src = G[0].astype(jnp.int32)
    dst = G[1].astype(jnp.int32)
    et = etypes.astype(jnp.int32)

    gidx = et * NPAD + src
    npad_e = EPAD - E
    pad_g = jnp.arange(npad_e, dtype=jnp.int32) % (R * NPAD)
    pad_d = N + jnp.arange(npad_e, dtype=jnp.int32) % (NACC - N)
    gidx = jnp.concatenate([gidx, pad_g])
    didx = jnp.concatenate([dst, pad_d])
    gidx = gidx.reshape(NW, K, C)
    didx = didx.reshape(NW, K, C)

    x0 = jnp.pad(emb, ((0, NPAD - N), (0, 0)))
    zeros = jnp.zeros((NPAD, D), jnp.float32)

    sc = _sc_edges_fn()
    table1 = _make_table(x0, V1, comp1)
    p1 = sc(table1, gidx, didx, zeros)
    table2, h1 = _combine_and_table(p1, x0, Wself1, b1, V2, comp2)
    p2 = sc(table2, gidx, didx, zeros)
    h2 = _combine(p2, h1, Wself2, b2, False)
    return h2[:N]
